# baseline (speedup 1.0000x reference)
import functools

import jax
import jax.numpy as jnp
import numpy as np
from jax import lax
from jax.experimental import pallas as pl
from jax.experimental.pallas import tpu as pltpu
from jax.experimental.pallas import tpu_sc as plsc

D = 128
LANES = 16
NCORES = 2
NSUB = 16
NW = NCORES * NSUB
B = 48
PMW = 144
N_ACC = 10224
RPT = N_ACC // NSUB
N_TBL = 10240
RB = 2048


def _u_body(x_ref, p_ref, wx_ref, wp_ref, b_ref, u_ref):
    i = pl.program_id(0)
    ny = N_TBL // RB
    qb = jnp.dot(p_ref[...], wp_ref[...], preferred_element_type=jnp.float32)

    @pl.when(i < ny)
    def _():
        u_ref[...] = (
            jnp.dot(x_ref[...], wx_ref[...],
                    preferred_element_type=jnp.float32) + qb + b_ref[...]
        ).astype(jnp.bfloat16)

    @pl.when(i >= ny)
    def _():
        u_ref[...] = qb.astype(jnp.bfloat16)


def _combine_body(a0_ref, a1_ref, o_ref):
    s = a0_ref[...] + a1_ref[...]
    o_ref[...] = s[:, :D] / (s[:, D:D + 1] + 1e-16)


def _make_sc_kernel(n_blocks):
    mesh = plsc.VectorSubcoreMesh(core_axis_name="c", subcore_axis_name="s")

    @functools.partial(
        pl.kernel,
        out_type=jax.ShapeDtypeStruct((NCORES, N_ACC, PMW), jnp.float32),
        mesh=mesh,
        scratch_types=[
            [pltpu.VMEM((2 * B,), jnp.int32)] * 2,
            [pltpu.VMEM((B,), jnp.int32)] * 2,
            [pltpu.VMEM((2 * B, D), jnp.bfloat16)] * 2,
            [pltpu.VMEM((B, PMW), jnp.float32)] * 2,
            pltpu.VMEM((D,), jnp.float32),
            pltpu.VMEM((LANES,), jnp.float32),
            pltpu.VMEM_SHARED((N_ACC, PMW), jnp.float32),
            [pltpu.SemaphoreType.DMA] * 2,
            [pltpu.SemaphoreType.DMA] * 2,
            [pltpu.SemaphoreType.DMA] * 2,
        ],
        compiler_params=pltpu.CompilerParams(
            needs_layout_passes=False, use_tc_tiling_on_sc=False),
    )
    def sc_kernel(u_hbm, sd_hbm, wg_hbm, bg_hbm, zrows_hbm,
                  out_hbm, sdv, dsc, ubuf, pmbuf, wgv, bgv, acc,
                  sem_i, sem_g, sem_sc):
        cid = lax.axis_index("c")
        sid = lax.axis_index("s")
        wid = cid * NSUB + sid

        pltpu.sync_copy(wg_hbm, wgv)
        pltpu.sync_copy(bg_hbm, bgv)
        pltpu.sync_copy(zrows_hbm, acc.at[pl.ds(sid * RPT, RPT)])
        plsc.subcore_barrier()

        bg = bgv[...]
        wvs = [wgv[pl.ds(LANES * j, LANES)] for j in range(D // LANES)]
        ntv = jnp.full((LANES,), N_TBL, jnp.int32)
        ibase = wid * n_blocks * 2 * B

        def issue_gather(s):
            pltpu.async_copy(u_hbm.at[sdv[s]], ubuf[s], sem_g[s])

        def drain_gather(s):
            pltpu.make_async_copy(u_hbm.at[sdv[s]], ubuf[s], sem_g[s]).wait()

        def drain_scatter(s):
            pltpu.make_async_copy(pmbuf[s], acc.at[dsc[s]], sem_sc[s]).wait()

        def compute_block(s):
            @plsc.parallel_loop(0, B, unroll=4)
            def _edge(e):
                ms = []
                dot = None
                for c in range(D // (2 * LANES)):
                    yv = ubuf[s][e, pl.ds(2 * LANES * c, 2 * LANES)]
                    qv = ubuf[s][B + e, pl.ds(2 * LANES * c, 2 * LANES)]
                    z = yv - qv
                    m = z / (1.0 + jnp.exp(-z))
                    ma, mb = plsc.unpack(
                        m, format=plsc.PackFormat.INTERLEAVED,
                        preferred_element_type=jnp.float32)
                    ms.append(ma)
                    ms.append(mb)
                    acc2 = ma * wvs[2 * c] + mb * wvs[2 * c + 1]
                    dot = acc2 if dot is None else dot + acc2
                t = jnp.sum(dot)
                g = jnp.broadcast_to(t, (LANES,)) + bg
                g = g / (1.0 + jnp.exp(-g))
                p = jnp.exp(g)
                for j in range(D // LANES):
                    pmbuf[s][e, pl.ds(LANES * j, LANES)] = p * ms[j]
                pmbuf[s][e, pl.ds(D, LANES)] = p

        for s in (0, 1):
            pltpu.sync_copy(sd_hbm.at[pl.ds(ibase + s * 2 * B, 2 * B)],
                            sdv[s])
        issue_gather(0)

        @pl.loop(0, n_blocks, step=2)
        def _blk(b):
            for s in (0, 1):
                bb = b + s

                @pl.when(bb >= 2)
                def _():
                    drain_scatter(s)

                drain_gather(s)
                for j in range(B // LANES):
                    dsc[s][pl.ds(LANES * j, LANES)] = (
                        sdv[s][pl.ds(B + LANES * j, LANES)] - ntv)

                @pl.when(bb + 2 < n_blocks)
                def _():
                    pltpu.async_copy(
                        sd_hbm.at[pl.ds(ibase + (bb + 2) * 2 * B, 2 * B)],
                        sdv[s], sem_i[s])

                @pl.when(bb + 1 < n_blocks)
                def _():
                    @pl.when(bb >= 1)
                    def _():
                        pltpu.make_async_copy(
                            sd_hbm.at[pl.ds(0, 2 * B)], sdv[1 - s],
                            sem_i[1 - s]).wait()
                    issue_gather(1 - s)

                compute_block(s)
                pltpu.async_copy(pmbuf[s], acc.at[dsc[s]], sem_sc[s],
                                 add=True)

        drain_scatter(0)
        drain_scatter(1)
        plsc.subcore_barrier()
        pltpu.sync_copy(acc.at[pl.ds(sid * RPT, RPT)],
                        out_hbm.at[cid, pl.ds(sid * RPT, RPT)])

    return sc_kernel


def kernel(x, pos, W_local, b_local, W_gate, b_gate, edge_index):
    n, d = x.shape
    e = edge_index.shape[1]
    etot = e + n
    ew = -(-etot // (NW * 2 * B)) * 2 * B
    n_blocks = ew // B
    epad = ew * NW

    perm = np.empty((d,), np.int32)
    for c in range(d // (2 * LANES)):
        for i in range(LANES):
            perm[32 * c + 2 * i] = 32 * c + i
            perm[32 * c + 2 * i + 1] = 32 * c + LANES + i

    xp = jnp.zeros((N_TBL, d), jnp.float32).at[:n].set(x)
    posp = jnp.zeros((N_TBL, 8), jnp.float32).at[:n, :3].set(pos)
    wx = W_local[:d][:, perm]
    wp = jnp.zeros((8, d), jnp.float32).at[:3].set(W_local[d:])[:, perm]
    bl = b_local.reshape(1, d)[:, perm]
    src = jnp.full((epad,), n, jnp.int32).at[:e].set(edge_index[0]).at[
        e:etot].set(jnp.arange(n, dtype=jnp.int32))
    dst = jnp.full((epad,), n, jnp.int32).at[:e].set(edge_index[1]).at[
        e:etot].set(jnp.arange(n, dtype=jnp.int32))
    sd = jnp.concatenate(
        [src.reshape(-1, B), dst.reshape(-1, B) + N_TBL], axis=1).reshape(-1)
    wg = W_gate[:, 0]
    bg16 = jnp.broadcast_to(b_gate, (LANES,)).astype(jnp.float32)
    zrows = jnp.zeros((RPT, PMW), jnp.float32)

    u = pl.pallas_call(
        _u_body,
        grid=(2 * N_TBL // RB,),
        in_specs=[
            pl.BlockSpec((RB, d), lambda i: (i % (N_TBL // RB), 0)),
            pl.BlockSpec((RB, 8), lambda i: (i % (N_TBL // RB), 0)),
            pl.BlockSpec((d, d), lambda i: (0, 0)),
            pl.BlockSpec((8, d), lambda i: (0, 0)),
            pl.BlockSpec((1, d), lambda i: (0, 0)),
        ],
        out_specs=pl.BlockSpec((RB, d), lambda i: (i, 0)),
        out_shape=jax.ShapeDtypeStruct((2 * N_TBL, d), jnp.bfloat16),
    )(xp, posp, wx, wp, bl)

    accs = _make_sc_kernel(n_blocks)(u, sd, wg, bg16, zrows)

    out = pl.pallas_call(
        _combine_body,
        grid=(pl.cdiv(N_ACC, RB),),
        in_specs=[
            pl.BlockSpec((RB, PMW), lambda i: (i, 0)),
            pl.BlockSpec((RB, PMW), lambda i: (i, 0)),
        ],
        out_specs=pl.BlockSpec((RB, d), lambda i: (i, 0)),
        out_shape=jax.ShapeDtypeStruct((N_ACC, d), jnp.float32),
    )(accs[0], accs[1])
    return out[:n]


# speedup vs baseline: 1.3230x; 1.3230x over previous
import functools

import jax
import jax.numpy as jnp
import numpy as np
from jax import lax
from jax.experimental import pallas as pl
from jax.experimental.pallas import tpu as pltpu
from jax.experimental.pallas import tpu_sc as plsc

D = 128
LANES = 16
NCORES = 2
NSUB = 16
NW = NCORES * NSUB
B = 64
PMW = 144
N_ACC = 10224
RPT = N_ACC // NSUB
N_TBL = 10240
RB = 2048


def _u_body(x_ref, p_ref, wx_ref, wp_ref, b_ref, u_ref):
    i = pl.program_id(0)
    ny = N_TBL // RB
    qb = jnp.dot(p_ref[...], wp_ref[...], preferred_element_type=jnp.float32)

    @pl.when(i < ny)
    def _():
        u_ref[...] = (
            jnp.dot(x_ref[...], wx_ref[...],
                    preferred_element_type=jnp.float32) + qb + b_ref[...]
        ).astype(jnp.bfloat16)

    @pl.when(i >= ny)
    def _():
        u_ref[...] = qb.astype(jnp.bfloat16)


def _combine_body(a0_ref, a1_ref, o_ref):
    s = a0_ref[...] + a1_ref[...]
    o_ref[...] = s[:, :D] / (s[:, D:D + 1] + 1e-16)


def _make_sc_kernel(n_blocks):
    mesh = plsc.VectorSubcoreMesh(core_axis_name="c", subcore_axis_name="s")

    @functools.partial(
        pl.kernel,
        out_type=jax.ShapeDtypeStruct((NCORES, N_ACC, PMW), jnp.float32),
        mesh=mesh,
        scratch_types=[
            [pltpu.VMEM((2 * B,), jnp.int32)] * 2,
            [pltpu.VMEM((B,), jnp.int32)] * 2,
            [pltpu.VMEM((2 * B, D), jnp.bfloat16)] * 2,
            [pltpu.VMEM((B, PMW), jnp.float32)] * 2,
            pltpu.VMEM((D,), jnp.float32),
            pltpu.VMEM((LANES,), jnp.float32),
            pltpu.VMEM_SHARED((N_ACC, PMW), jnp.float32),
            [pltpu.SemaphoreType.DMA] * 2,
            [pltpu.SemaphoreType.DMA] * 2,
            [pltpu.SemaphoreType.DMA] * 2,
        ],
        compiler_params=pltpu.CompilerParams(
            needs_layout_passes=False, use_tc_tiling_on_sc=False),
    )
    def sc_kernel(u_hbm, sd_hbm, wg_hbm, bg_hbm, zrows_hbm,
                  out_hbm, sdv, dsc, ubuf, pmbuf, wgv, bgv, acc,
                  sem_i, sem_g, sem_sc):
        cid = lax.axis_index("c")
        sid = lax.axis_index("s")
        wid = cid * NSUB + sid

        pltpu.sync_copy(wg_hbm, wgv)
        pltpu.sync_copy(bg_hbm, bgv)
        pltpu.sync_copy(zrows_hbm, acc.at[pl.ds(sid * RPT, RPT)])
        plsc.subcore_barrier()

        bg = bgv[...]
        wvs = [wgv[pl.ds(LANES * j, LANES)] for j in range(D // LANES)]
        ntv = jnp.full((LANES,), N_TBL, jnp.int32)
        ibase = wid * n_blocks * 2 * B

        def issue_gather(s):
            pltpu.async_copy(u_hbm.at[sdv[s]], ubuf[s], sem_g[s])

        def drain_gather(s):
            pltpu.make_async_copy(u_hbm.at[sdv[s]], ubuf[s], sem_g[s]).wait()

        def drain_scatter(s):
            pltpu.make_async_copy(pmbuf[s], acc.at[dsc[s]], sem_sc[s]).wait()

        def compute_block(s):
            @plsc.parallel_loop(0, B, unroll=3)
            def _edge(e):
                ms = []
                dot = None
                for c in range(D // (2 * LANES)):
                    yv = ubuf[s][e, pl.ds(2 * LANES * c, 2 * LANES)]
                    qv = ubuf[s][B + e, pl.ds(2 * LANES * c, 2 * LANES)]
                    z = yv - qv
                    m = z / (1.0 + jnp.exp(-z))
                    ma, mb = plsc.unpack(
                        m, format=plsc.PackFormat.INTERLEAVED,
                        preferred_element_type=jnp.float32)
                    ms.append(ma)
                    ms.append(mb)
                    acc2 = ma * wvs[2 * c] + mb * wvs[2 * c + 1]
                    dot = acc2 if dot is None else dot + acc2
                t = jnp.sum(dot)
                g = jnp.broadcast_to(t, (LANES,)) + bg
                g = g / (1.0 + jnp.exp(-g))
                p = jnp.exp(g)
                for j in range(D // LANES):
                    pmbuf[s][e, pl.ds(LANES * j, LANES)] = p * ms[j]
                pmbuf[s][e, pl.ds(D, LANES)] = p

        for s in (0, 1):
            pltpu.sync_copy(sd_hbm.at[pl.ds(ibase + s * 2 * B, 2 * B)],
                            sdv[s])
        issue_gather(0)

        @pl.loop(0, n_blocks, step=2)
        def _blk(b):
            for s in (0, 1):
                bb = b + s

                @pl.when(bb >= 2)
                def _():
                    drain_scatter(s)

                drain_gather(s)
                for j in range(B // LANES):
                    dsc[s][pl.ds(LANES * j, LANES)] = (
                        sdv[s][pl.ds(B + LANES * j, LANES)] - ntv)

                @pl.when(bb + 2 < n_blocks)
                def _():
                    pltpu.async_copy(
                        sd_hbm.at[pl.ds(ibase + (bb + 2) * 2 * B, 2 * B)],
                        sdv[s], sem_i[s])

                @pl.when(bb + 1 < n_blocks)
                def _():
                    @pl.when(bb >= 1)
                    def _():
                        pltpu.make_async_copy(
                            sd_hbm.at[pl.ds(0, 2 * B)], sdv[1 - s],
                            sem_i[1 - s]).wait()
                    issue_gather(1 - s)

                compute_block(s)
                pltpu.async_copy(pmbuf[s], acc.at[dsc[s]], sem_sc[s],
                                 add=True)

        drain_scatter(0)
        drain_scatter(1)
        plsc.subcore_barrier()
        pltpu.sync_copy(acc.at[pl.ds(sid * RPT, RPT)],
                        out_hbm.at[cid, pl.ds(sid * RPT, RPT)])

    return sc_kernel


def kernel(x, pos, W_local, b_local, W_gate, b_gate, edge_index):
    n, d = x.shape
    e = edge_index.shape[1]
    etot = e + n
    ew = -(-etot // (NW * 2 * B)) * 2 * B
    n_blocks = ew // B
    epad = ew * NW

    perm = np.empty((d,), np.int32)
    for c in range(d // (2 * LANES)):
        for i in range(LANES):
            perm[32 * c + 2 * i] = 32 * c + i
            perm[32 * c + 2 * i + 1] = 32 * c + LANES + i

    xp = jnp.zeros((N_TBL, d), jnp.float32).at[:n].set(x)
    posp = jnp.zeros((N_TBL, 8), jnp.float32).at[:n, :3].set(pos)
    wx = W_local[:d][:, perm]
    wp = jnp.zeros((8, d), jnp.float32).at[:3].set(W_local[d:])[:, perm]
    bl = b_local.reshape(1, d)[:, perm]
    src = jnp.full((epad,), n, jnp.int32).at[:e].set(edge_index[0]).at[
        e:etot].set(jnp.arange(n, dtype=jnp.int32))
    dst = jnp.full((epad,), n, jnp.int32).at[:e].set(edge_index[1]).at[
        e:etot].set(jnp.arange(n, dtype=jnp.int32))
    sd = jnp.concatenate(
        [src.reshape(-1, B), dst.reshape(-1, B) + N_TBL], axis=1).reshape(-1)
    wg = W_gate[:, 0]
    bg16 = jnp.broadcast_to(b_gate, (LANES,)).astype(jnp.float32)
    zrows = jnp.zeros((RPT, PMW), jnp.float32)

    u = pl.pallas_call(
        _u_body,
        grid=(2 * N_TBL // RB,),
        in_specs=[
            pl.BlockSpec((RB, d), lambda i: (i % (N_TBL // RB), 0)),
            pl.BlockSpec((RB, 8), lambda i: (i % (N_TBL // RB), 0)),
            pl.BlockSpec((d, d), lambda i: (0, 0)),
            pl.BlockSpec((8, d), lambda i: (0, 0)),
            pl.BlockSpec((1, d), lambda i: (0, 0)),
        ],
        out_specs=pl.BlockSpec((RB, d), lambda i: (i, 0)),
        out_shape=jax.ShapeDtypeStruct((2 * N_TBL, d), jnp.bfloat16),
    )(xp, posp, wx, wp, bl)

    accs = _make_sc_kernel(n_blocks)(u, sd, wg, bg16, zrows)

    out = pl.pallas_call(
        _combine_body,
        grid=(pl.cdiv(N_ACC, RB),),
        in_specs=[
            pl.BlockSpec((RB, PMW), lambda i: (i, 0)),
            pl.BlockSpec((RB, PMW), lambda i: (i, 0)),
        ],
        out_specs=pl.BlockSpec((RB, d), lambda i: (i, 0)),
        out_shape=jax.ShapeDtypeStruct((N_ACC, d), jnp.float32),
    )(accs[0], accs[1])
    return out[:n]
